# baseline (speedup 1.0000x reference)
import functools

import jax
import jax.numpy as jnp
from jax import lax
from jax.experimental import pallas as pl
from jax.experimental.pallas import tpu as pltpu
from jax.experimental.pallas import tpu_sc as plsc

N_NODES = 10000
N_EDGES = 320000
D = 128
DH = 64
D_EDGE = 16
EPS = 1e-07

NC = 2
NS = 16
CHUNK = 64
NCH = 316
EPT = NCH * CHUNK
NE_P = EPT * NS
PAD = NE_P - N_EDGES

ROW_BLK = 64
ROW_BLKS = 10
ACC_ROWS = ROW_BLK * ROW_BLKS * NS


def _ea_body(attr_ref, w0_ref, w1_ref, o0_ref, o1_ref):
    a = attr_ref[...]
    o0_ref[...] = jnp.dot(a, w0_ref[...], preferred_element_type=jnp.float32)
    o1_ref[...] = jnp.dot(a, w1_ref[...], preferred_element_type=jnp.float32)


def _ea_call(attr2, wbd0, wbd1):
    blk = 8000
    grid = (N_EDGES // 2 // blk,)
    return pl.pallas_call(
        _ea_body,
        grid=grid,
        in_specs=[
            pl.BlockSpec((blk, 2 * D_EDGE), lambda i: (i, 0)),
            pl.BlockSpec((2 * D_EDGE, D), lambda i: (0, 0)),
            pl.BlockSpec((2 * D_EDGE, D), lambda i: (0, 0)),
        ],
        out_specs=[
            pl.BlockSpec((blk, D), lambda i: (i, 0)),
            pl.BlockSpec((blk, D), lambda i: (i, 0)),
        ],
        out_shape=[
            jax.ShapeDtypeStruct((NE_P // 2, D), jnp.float32),
            jax.ShapeDtypeStruct((NE_P // 2, D), jnp.float32),
        ],
    )(attr2, wbd0, wbd1)


def _sc_body(src_hbm, dst_hbm, x_hbm, ea0_hbm, ea1_hbm, z_hbm,
             agg0_hbm, agg1_hbm, acc,
             is0, is1, id0, id1, id2, id3, xb0, xb1, eb0, eb1, mw0, mw1,
             sis0, sis1, sid0, sid1, sid2, sid3, sg0, sg1, se0, se1,
             ssc0, ssc1):
    c = lax.axis_index("c")
    s = lax.axis_index("s")

    isb = (is0, is1)
    idb = (id0, id1, id2, id3)
    xbb = (xb0, xb1)
    ebb = (eb0, eb1)
    mwb = (mw0, mw1)
    sis = (sis0, sis1)
    sid = (sid0, sid1, sid2, sid3)
    sg = (sg0, sg1)
    se = (se0, se1)
    ssc = (ssc0, ssc1)

    @pl.when(s == 0)
    def _():
        pltpu.sync_copy(z_hbm, acc)

    plsc.subcore_barrier()

    def edge_pass(col0, ea_hbm):
        ebase = s * EPT

        def src_sl(g):
            return src_hbm.at[pl.ds(ebase + g * CHUNK, CHUNK)]

        def dst_sl(g):
            return dst_hbm.at[pl.ds(ebase + g * CHUNK, CHUNK)]

        def ea_sl(g):
            return ea_hbm.at[pl.ds(s * (EPT // 2) + g * (CHUNK // 2),
                                   CHUNK // 2)]

        def x_gat(idx_ref):
            return x_hbm.at[idx_ref]

        pltpu.async_copy(src_sl(0), is0, sis0)
        pltpu.async_copy(src_sl(1), is1, sis1)
        pltpu.async_copy(dst_sl(0), id0, sid0)
        pltpu.async_copy(dst_sl(1), id1, sid1)
        pltpu.make_async_copy(src_sl(0), is0, sis0).wait()
        pltpu.async_copy(x_gat(is0), xb0, sg0)
        pltpu.async_copy(ea_sl(0), eb0, se0)

        def quad_body(p, carry):
            for b in range(4):
                g = 4 * p + b
                m2 = b & 1
                n2 = 1 - m2
                gn = jnp.minimum(g + 1, NCH - 1)
                g2 = jnp.minimum(g + 2, NCH - 1)
                pltpu.make_async_copy(src_sl(gn), isb[n2], sis[n2]).wait()
                pltpu.async_copy(x_gat(isb[n2]), xbb[n2], sg[n2])
                pltpu.async_copy(ea_sl(gn), ebb[n2], se[n2])
                pltpu.make_async_copy(x_gat(isb[m2]), xbb[m2],
                                      sg[m2]).wait()
                pltpu.make_async_copy(ea_sl(g), ebb[m2], se[m2]).wait()
                pltpu.async_copy(src_sl(g2), isb[m2], sis[m2])

                @pl.when(g >= 2)
                def _():
                    pltpu.make_async_copy(mwb[m2], acc.at[idb[(b + 2) % 4]],
                                          ssc[m2]).wait()

                pltpu.async_copy(dst_sl(g2), idb[(b + 2) % 4],
                                 sid[(b + 2) % 4])

                @plsc.parallel_loop(0, CHUNK // 2, step=1, unroll=2)
                def _(r2):
                    for h in range(2):
                        for j in range(DH // 16):
                            xv = xbb[m2][2 * r2 + h, pl.ds(col0 + j * 16, 16)]
                            ev = ebb[m2][r2, pl.ds(h * DH + j * 16, 16)]
                            m = jnp.maximum(xv + ev, 0.0) + EPS
                            w = jnp.exp(m)
                            mwb[m2][2 * r2 + h, pl.ds(j * 16, 16)] = m * w
                            mwb[m2][2 * r2 + h, pl.ds(DH + j * 16, 16)] = w

                pltpu.make_async_copy(dst_sl(g), idb[b], sid[b]).wait()
                pltpu.async_copy(mwb[m2], acc.at[idb[b]], ssc[m2], add=True)
            return carry

        lax.fori_loop(0, NCH // 4, quad_body, 0)

        pltpu.make_async_copy(x_gat(isb[0]), xbb[0], sg[0]).wait()
        pltpu.make_async_copy(ea_sl(NCH - 1), ebb[0], se[0]).wait()
        pltpu.make_async_copy(src_sl(NCH - 1), isb[1], sis[1]).wait()
        pltpu.make_async_copy(dst_sl(NCH - 1), idb[0], sid[0]).wait()
        pltpu.make_async_copy(dst_sl(NCH - 1), idb[1], sid[1]).wait()
        pltpu.make_async_copy(mwb[0], acc.at[idb[2]], ssc[0]).wait()
        pltpu.make_async_copy(mwb[1], acc.at[idb[3]], ssc[1]).wait()

    @pl.when(c == 0)
    def _():
        edge_pass(0, ea0_hbm)

    @pl.when(c == 1)
    def _():
        edge_pass(DH, ea1_hbm)

    plsc.subcore_barrier()

    def div_pass(agg_hbm):
        def blk_body(blk, carry):
            row0 = s * (ROW_BLK * ROW_BLKS) + blk * ROW_BLK
            pltpu.sync_copy(acc.at[pl.ds(row0, ROW_BLK)], mw0)

            @plsc.parallel_loop(0, ROW_BLK // 2, step=1, unroll=2)
            def _(r2):
                for h in range(2):
                    for j in range(DH // 16):
                        num = mw0[2 * r2 + h, pl.ds(j * 16, 16)]
                        den = mw0[2 * r2 + h, pl.ds(DH + j * 16, 16)]
                        eb0[r2, pl.ds(h * DH + j * 16, 16)] = (
                            num / (den + 1e-16))

            row0h = (s * (ROW_BLK * ROW_BLKS // 2) + blk * (ROW_BLK // 2))
            pltpu.sync_copy(eb0, agg_hbm.at[pl.ds(row0h, ROW_BLK // 2)])
            return carry

        lax.fori_loop(0, ROW_BLKS, blk_body, 0)

    @pl.when(c == 0)
    def _():
        div_pass(agg0_hbm)

    @pl.when(c == 1)
    def _():
        div_pass(agg1_hbm)


def _sc_call(src_p, dst_p, x, ea0, ea1, zeros):
    mesh = plsc.VectorSubcoreMesh(
        core_axis_name="c", subcore_axis_name="s", num_cores=NC,
        num_subcores=NS)
    f = functools.partial(
        pl.kernel,
        out_type=(
            jax.ShapeDtypeStruct((ACC_ROWS // 2, D), jnp.float32),
            jax.ShapeDtypeStruct((ACC_ROWS // 2, D), jnp.float32),
        ),
        mesh=mesh,
        scratch_types=[
            pltpu.VMEM_SHARED((ACC_ROWS, D), jnp.float32),
            pltpu.VMEM((CHUNK,), jnp.int32),
            pltpu.VMEM((CHUNK,), jnp.int32),
            pltpu.VMEM((CHUNK,), jnp.int32),
            pltpu.VMEM((CHUNK,), jnp.int32),
            pltpu.VMEM((CHUNK,), jnp.int32),
            pltpu.VMEM((CHUNK,), jnp.int32),
            pltpu.VMEM((CHUNK, D), jnp.float32),
            pltpu.VMEM((CHUNK, D), jnp.float32),
            pltpu.VMEM((CHUNK // 2, D), jnp.float32),
            pltpu.VMEM((CHUNK // 2, D), jnp.float32),
            pltpu.VMEM((CHUNK, D), jnp.float32),
            pltpu.VMEM((CHUNK, D), jnp.float32),
        ] + [pltpu.SemaphoreType.DMA] * 12,
    )(_sc_body)
    return f(src_p, dst_p, x, ea0, ea1, zeros)


def _mlp_body(x0_ref, x1_ref, a0_ref, a1_ref, w1a_ref, w1b_ref,
              g_ref, b_ref, w2_ref, o_ref):
    h0 = a0_ref[...] + x0_ref[...]
    h1 = a1_ref[...] + x1_ref[...]
    z = (jnp.dot(h0, w1a_ref[...], preferred_element_type=jnp.float32)
         + jnp.dot(h1, w1b_ref[...], preferred_element_type=jnp.float32))
    mean = jnp.mean(z, axis=0, keepdims=True)
    zc = z - mean
    var = jnp.mean(zc * zc, axis=0, keepdims=True)
    zn = zc * lax.rsqrt(var + 1e-5) * g_ref[...] + b_ref[...]
    zr = jnp.maximum(zn, 0.0)
    o_ref[...] = jnp.dot(zr, w2_ref[...], preferred_element_type=jnp.float32)


def _mlp_call(x0, x1, a0, a1, w1a, w1b, gamma, beta, w2):
    return pl.pallas_call(
        _mlp_body,
        out_shape=jax.ShapeDtypeStruct((N_NODES, D), jnp.float32),
    )(x0, x1, a0, a1, w1a, w1b, gamma.reshape(1, 2 * D),
      beta.reshape(1, 2 * D), w2)


def kernel(x, edge_index, edge_attr, W_edge, W1, gamma, beta, W2):
    src = edge_index[0].astype(jnp.int32)
    dst = edge_index[1].astype(jnp.int32)
    src_p = jnp.concatenate([src, jnp.zeros((PAD,), jnp.int32)])
    dst_p = jnp.concatenate([dst, jnp.full((PAD,), N_NODES, jnp.int32)])
    x0 = x[:, :DH]
    x1 = x[:, DH:]
    zeros = jnp.zeros((ACC_ROWS, D), jnp.float32)

    attr2 = edge_attr.reshape(N_EDGES // 2, 2 * D_EDGE)
    zde = jnp.zeros((D_EDGE, DH), jnp.float32)
    wbd0 = jnp.concatenate(
        [jnp.concatenate([W_edge[:, :DH], zde], axis=1),
         jnp.concatenate([zde, W_edge[:, :DH]], axis=1)], axis=0)
    wbd1 = jnp.concatenate(
        [jnp.concatenate([W_edge[:, DH:], zde], axis=1),
         jnp.concatenate([zde, W_edge[:, DH:]], axis=1)], axis=0)

    ea0, ea1 = _ea_call(attr2, wbd0, wbd1)
    agg0p, agg1p = _sc_call(src_p, dst_p, x, ea0, ea1, zeros)
    agg0 = agg0p.reshape(ACC_ROWS, DH)[:N_NODES]
    agg1 = agg1p.reshape(ACC_ROWS, DH)[:N_NODES]
    return _mlp_call(x0, x1, agg0, agg1,
                     W1[:DH], W1[DH:], gamma, beta, W2)


# speedup vs baseline: 1.0619x; 1.0619x over previous
import functools

import jax
import jax.numpy as jnp
from jax import lax
from jax.experimental import pallas as pl
from jax.experimental.pallas import tpu as pltpu
from jax.experimental.pallas import tpu_sc as plsc

N_NODES = 10000
N_EDGES = 320000
D = 128
DH = 64
D_EDGE = 16
EPS = 1e-07

NC = 2
NS = 16
CHUNK = 64
NCH = 316
EPT = NCH * CHUNK
NE_P = EPT * NS
PAD = NE_P - N_EDGES

ROW_BLK = 64
ROW_BLKS = 10
ACC_ROWS = ROW_BLK * ROW_BLKS * NS


def _ea_body(attr_ref, w_ref, o_ref):
    a = attr_ref[...]
    o_ref[...] = jnp.dot(a, w_ref[...], preferred_element_type=jnp.float32)


def _ea_call(attr, we):
    blk = 8000
    grid = (N_EDGES // blk,)
    return pl.pallas_call(
        _ea_body,
        grid=grid,
        in_specs=[
            pl.BlockSpec((blk, D_EDGE), lambda i: (i, 0)),
            pl.BlockSpec((D_EDGE, D), lambda i: (0, 0)),
        ],
        out_specs=pl.BlockSpec((blk, D), lambda i: (i, 0)),
        out_shape=jax.ShapeDtypeStruct((NE_P, D), jnp.float32),
    )(attr, we)


def _sc_body(src_hbm, dst_hbm, x_hbm, ea_hbm, z_hbm,
             agg0_hbm, agg1_hbm, acc,
             is0, is1, id0, id1, id2, id3, xb0, xb1, eb0, eb1, mw0, mw1,
             sis0, sis1, sid0, sid1, sid2, sid3, sg0, sg1, se0, se1,
             ssc0, ssc1):
    c = lax.axis_index("c")
    s = lax.axis_index("s")

    isb = (is0, is1)
    idb = (id0, id1, id2, id3)
    xbb = (xb0, xb1)
    ebb = (eb0, eb1)
    mwb = (mw0, mw1)
    sis = (sis0, sis1)
    sid = (sid0, sid1, sid2, sid3)
    sg = (sg0, sg1)
    se = (se0, se1)
    ssc = (ssc0, ssc1)

    @pl.when(s == 0)
    def _():
        pltpu.sync_copy(z_hbm, acc)

    plsc.subcore_barrier()

    def edge_pass(col0):
        ebase = s * EPT

        def src_sl(g):
            return src_hbm.at[pl.ds(ebase + g * CHUNK, CHUNK)]

        def dst_sl(g):
            return dst_hbm.at[pl.ds(ebase + g * CHUNK, CHUNK)]

        def ea_sl(g):
            return ea_hbm.at[pl.ds(ebase + g * CHUNK, CHUNK),
                             pl.ds(col0, DH)]

        def x_gat(idx_ref):
            return x_hbm.at[idx_ref]

        pltpu.async_copy(src_sl(0), is0, sis0)
        pltpu.async_copy(src_sl(1), is1, sis1)
        pltpu.async_copy(dst_sl(0), id0, sid0)
        pltpu.async_copy(dst_sl(1), id1, sid1)
        pltpu.make_async_copy(src_sl(0), is0, sis0).wait()
        pltpu.async_copy(x_gat(is0), xb0, sg0)
        pltpu.async_copy(ea_sl(0), eb0, se0)

        def quad_body(p, carry):
            for b in range(4):
                g = 4 * p + b
                m2 = b & 1
                n2 = 1 - m2
                gn = jnp.minimum(g + 1, NCH - 1)
                g2 = jnp.minimum(g + 2, NCH - 1)
                pltpu.make_async_copy(src_sl(gn), isb[n2], sis[n2]).wait()
                pltpu.async_copy(x_gat(isb[n2]), xbb[n2], sg[n2])
                pltpu.async_copy(ea_sl(gn), ebb[n2], se[n2])
                pltpu.make_async_copy(x_gat(isb[m2]), xbb[m2],
                                      sg[m2]).wait()
                pltpu.make_async_copy(ea_sl(g), ebb[m2], se[m2]).wait()
                pltpu.async_copy(src_sl(g2), isb[m2], sis[m2])

                @pl.when(g >= 2)
                def _():
                    pltpu.make_async_copy(mwb[m2], acc.at[idb[(b + 2) % 4]],
                                          ssc[m2]).wait()

                pltpu.async_copy(dst_sl(g2), idb[(b + 2) % 4],
                                 sid[(b + 2) % 4])

                @plsc.parallel_loop(0, CHUNK, step=1, unroll=4)
                def _(r):
                    for j in range(DH // 16):
                        xv = xbb[m2][r, pl.ds(col0 + j * 16, 16)]
                        ev = ebb[m2][r, pl.ds(j * 16, 16)]
                        m = jnp.maximum(xv + ev, 0.0) + EPS
                        w = jnp.exp(m)
                        mwb[m2][r, pl.ds(j * 16, 16)] = m * w
                        mwb[m2][r, pl.ds(DH + j * 16, 16)] = w

                pltpu.make_async_copy(dst_sl(g), idb[b], sid[b]).wait()
                pltpu.async_copy(mwb[m2], acc.at[idb[b]], ssc[m2], add=True)
            return carry

        lax.fori_loop(0, NCH // 4, quad_body, 0)

        pltpu.make_async_copy(x_gat(isb[0]), xbb[0], sg[0]).wait()
        pltpu.make_async_copy(ea_sl(NCH - 1), ebb[0], se[0]).wait()
        pltpu.make_async_copy(src_sl(NCH - 1), isb[1], sis[1]).wait()
        pltpu.make_async_copy(dst_sl(NCH - 1), idb[0], sid[0]).wait()
        pltpu.make_async_copy(dst_sl(NCH - 1), idb[1], sid[1]).wait()
        pltpu.make_async_copy(mwb[0], acc.at[idb[2]], ssc[0]).wait()
        pltpu.make_async_copy(mwb[1], acc.at[idb[3]], ssc[1]).wait()

    @pl.when(c == 0)
    def _():
        edge_pass(0)

    @pl.when(c == 1)
    def _():
        edge_pass(DH)

    plsc.subcore_barrier()

    def div_pass(agg_hbm):
        def blk_body(blk, carry):
            row0 = s * (ROW_BLK * ROW_BLKS) + blk * ROW_BLK
            pltpu.sync_copy(acc.at[pl.ds(row0, ROW_BLK)], mw0)

            @plsc.parallel_loop(0, ROW_BLK, step=1, unroll=4)
            def _(r):
                for j in range(DH // 16):
                    num = mw0[r, pl.ds(j * 16, 16)]
                    den = mw0[r, pl.ds(DH + j * 16, 16)]
                    eb0[r, pl.ds(j * 16, 16)] = num / (den + 1e-16)
            pltpu.sync_copy(eb0, agg_hbm.at[pl.ds(row0, ROW_BLK)])
            return carry

        lax.fori_loop(0, ROW_BLKS, blk_body, 0)

    @pl.when(c == 0)
    def _():
        div_pass(agg0_hbm)

    @pl.when(c == 1)
    def _():
        div_pass(agg1_hbm)


def _sc_call(src_p, dst_p, x, ea, zeros):
    mesh = plsc.VectorSubcoreMesh(
        core_axis_name="c", subcore_axis_name="s", num_cores=NC,
        num_subcores=NS)
    f = functools.partial(
        pl.kernel,
        out_type=(
            jax.ShapeDtypeStruct((ACC_ROWS, DH), jnp.float32),
            jax.ShapeDtypeStruct((ACC_ROWS, DH), jnp.float32),
        ),
        mesh=mesh,
        compiler_params=pltpu.CompilerParams(use_tc_tiling_on_sc=False),
        scratch_types=[
            pltpu.VMEM_SHARED((ACC_ROWS, D), jnp.float32),
            pltpu.VMEM((CHUNK,), jnp.int32),
            pltpu.VMEM((CHUNK,), jnp.int32),
            pltpu.VMEM((CHUNK,), jnp.int32),
            pltpu.VMEM((CHUNK,), jnp.int32),
            pltpu.VMEM((CHUNK,), jnp.int32),
            pltpu.VMEM((CHUNK,), jnp.int32),
            pltpu.VMEM((CHUNK, D), jnp.float32),
            pltpu.VMEM((CHUNK, D), jnp.float32),
            pltpu.VMEM((CHUNK, DH), jnp.float32),
            pltpu.VMEM((CHUNK, DH), jnp.float32),
            pltpu.VMEM((CHUNK, D), jnp.float32),
            pltpu.VMEM((CHUNK, D), jnp.float32),
        ] + [pltpu.SemaphoreType.DMA] * 12,
    )(_sc_body)
    return f(src_p, dst_p, x, ea, zeros)


def _mlp_body(x0_ref, x1_ref, a0_ref, a1_ref, w1a_ref, w1b_ref,
              g_ref, b_ref, w2_ref, o_ref):
    h0 = a0_ref[...] + x0_ref[...]
    h1 = a1_ref[...] + x1_ref[...]
    z = (jnp.dot(h0, w1a_ref[...], preferred_element_type=jnp.float32)
         + jnp.dot(h1, w1b_ref[...], preferred_element_type=jnp.float32))
    mean = jnp.mean(z, axis=0, keepdims=True)
    zc = z - mean
    var = jnp.mean(zc * zc, axis=0, keepdims=True)
    zn = zc * lax.rsqrt(var + 1e-5) * g_ref[...] + b_ref[...]
    zr = jnp.maximum(zn, 0.0)
    o_ref[...] = jnp.dot(zr, w2_ref[...], preferred_element_type=jnp.float32)


def _mlp_call(x0, x1, a0, a1, w1a, w1b, gamma, beta, w2):
    return pl.pallas_call(
        _mlp_body,
        out_shape=jax.ShapeDtypeStruct((N_NODES, D), jnp.float32),
    )(x0, x1, a0, a1, w1a, w1b, gamma.reshape(1, 2 * D),
      beta.reshape(1, 2 * D), w2)


def kernel(x, edge_index, edge_attr, W_edge, W1, gamma, beta, W2):
    src = edge_index[0].astype(jnp.int32)
    dst = edge_index[1].astype(jnp.int32)
    src_p = jnp.concatenate([src, jnp.zeros((PAD,), jnp.int32)])
    dst_p = jnp.concatenate([dst, jnp.full((PAD,), N_NODES, jnp.int32)])
    x0 = x[:, :DH]
    x1 = x[:, DH:]
    zeros = jnp.zeros((ACC_ROWS, D), jnp.float32)

    ea = _ea_call(edge_attr, W_edge)
    agg0, agg1 = _sc_call(src_p, dst_p, x, ea, zeros)
    return _mlp_call(x0, x1, agg0[:N_NODES], agg1[:N_NODES],
                     W1[:DH], W1[DH:], gamma, beta, W2)


# speedup vs baseline: 1.4147x; 1.3322x over previous
import functools

import jax
import jax.numpy as jnp
from jax import lax
from jax.experimental import pallas as pl
from jax.experimental.pallas import tpu as pltpu
from jax.experimental.pallas import tpu_sc as plsc

N_NODES = 10000
N_EDGES = 320000
D = 128
DH = 64
D_EDGE = 16
EPS = 1e-07

NC = 2
NS = 16
CHUNK = 64
NCH = 316
EPT = NCH * CHUNK
NE_P = EPT * NS
PAD = NE_P - N_EDGES

ROW_BLK = 64
ROW_BLKS = 10
ACC_ROWS = ROW_BLK * ROW_BLKS * NS


def _ea_body(attr_ref, w_ref, o_ref):
    a = attr_ref[...]
    o_ref[...] = jnp.dot(a, w_ref[...], preferred_element_type=jnp.float32)


def _ea_call(attr, we):
    blk = 8000
    grid = (N_EDGES // blk,)
    return pl.pallas_call(
        _ea_body,
        grid=grid,
        in_specs=[
            pl.BlockSpec((blk, D_EDGE), lambda i: (i, 0)),
            pl.BlockSpec((D_EDGE, D), lambda i: (0, 0)),
        ],
        out_specs=pl.BlockSpec((blk, D), lambda i: (i, 0)),
        out_shape=jax.ShapeDtypeStruct((NE_P, D), jnp.float32),
    )(attr, we)


def _sc_body(src_hbm, dst_hbm, x0_hbm, x1_hbm, ea_hbm, z_hbm,
             agg0_hbm, agg1_hbm, acc,
             is0, is1, id0, id1, id2, id3, xb0, xb1, eb0, eb1, mw0, mw1,
             sis0, sis1, sid0, sid1, sid2, sid3, sg0, sg1, se0, se1,
             ssc0, ssc1):
    c = lax.axis_index("c")
    s = lax.axis_index("s")

    isb = (is0, is1)
    idb = (id0, id1, id2, id3)
    xbb = (xb0, xb1)
    ebb = (eb0, eb1)
    mwb = (mw0, mw1)
    sis = (sis0, sis1)
    sid = (sid0, sid1, sid2, sid3)
    sg = (sg0, sg1)
    se = (se0, se1)
    ssc = (ssc0, ssc1)

    @pl.when(s == 0)
    def _():
        pltpu.sync_copy(z_hbm, acc)

    plsc.subcore_barrier()

    def edge_pass(col0, x_tab):
        ebase = s * EPT

        def src_sl(g):
            return src_hbm.at[pl.ds(ebase + g * CHUNK, CHUNK)]

        def dst_sl(g):
            return dst_hbm.at[pl.ds(ebase + g * CHUNK, CHUNK)]

        def ea_sl(g):
            return ea_hbm.at[pl.ds(ebase + g * CHUNK, CHUNK),
                             pl.ds(col0, DH)]

        def x_gat(idx_ref):
            return x_tab.at[idx_ref]

        pltpu.async_copy(src_sl(0), is0, sis0)
        pltpu.async_copy(src_sl(1), is1, sis1)
        pltpu.async_copy(dst_sl(0), id0, sid0)
        pltpu.async_copy(dst_sl(1), id1, sid1)
        pltpu.make_async_copy(src_sl(0), is0, sis0).wait()
        pltpu.async_copy(x_gat(is0), xb0, sg0)
        pltpu.async_copy(ea_sl(0), eb0, se0)

        def quad_body(p, carry):
            for b in range(4):
                g = 4 * p + b
                m2 = b & 1
                n2 = 1 - m2
                gn = jnp.minimum(g + 1, NCH - 1)
                g2 = jnp.minimum(g + 2, NCH - 1)
                pltpu.make_async_copy(src_sl(gn), isb[n2], sis[n2]).wait()
                pltpu.async_copy(x_gat(isb[n2]), xbb[n2], sg[n2])
                pltpu.async_copy(ea_sl(gn), ebb[n2], se[n2])
                pltpu.make_async_copy(x_gat(isb[m2]), xbb[m2],
                                      sg[m2]).wait()
                pltpu.make_async_copy(ea_sl(g), ebb[m2], se[m2]).wait()
                pltpu.async_copy(src_sl(g2), isb[m2], sis[m2])

                @pl.when(g >= 2)
                def _():
                    pltpu.make_async_copy(mwb[m2], acc.at[idb[(b + 2) % 4]],
                                          ssc[m2]).wait()

                pltpu.async_copy(dst_sl(g2), idb[(b + 2) % 4],
                                 sid[(b + 2) % 4])

                @plsc.parallel_loop(0, CHUNK, step=1, unroll=4)
                def _(r):
                    for j in range(DH // 16):
                        xv = xbb[m2][r, pl.ds(j * 16, 16)]
                        ev = ebb[m2][r, pl.ds(j * 16, 16)]
                        m = jnp.maximum(xv + ev, 0.0) + EPS
                        w = jnp.exp(m)
                        mwb[m2][r, pl.ds(j * 16, 16)] = m * w
                        mwb[m2][r, pl.ds(DH + j * 16, 16)] = w

                pltpu.make_async_copy(dst_sl(g), idb[b], sid[b]).wait()
                pltpu.async_copy(mwb[m2], acc.at[idb[b]], ssc[m2], add=True)
            return carry

        lax.fori_loop(0, NCH // 4, quad_body, 0)

        pltpu.make_async_copy(x_gat(isb[0]), xbb[0], sg[0]).wait()
        pltpu.make_async_copy(ea_sl(NCH - 1), ebb[0], se[0]).wait()
        pltpu.make_async_copy(src_sl(NCH - 1), isb[1], sis[1]).wait()
        pltpu.make_async_copy(dst_sl(NCH - 1), idb[0], sid[0]).wait()
        pltpu.make_async_copy(dst_sl(NCH - 1), idb[1], sid[1]).wait()
        pltpu.make_async_copy(mwb[0], acc.at[idb[2]], ssc[0]).wait()
        pltpu.make_async_copy(mwb[1], acc.at[idb[3]], ssc[1]).wait()

    @pl.when(c == 0)
    def _():
        edge_pass(0, x0_hbm)

    @pl.when(c == 1)
    def _():
        edge_pass(DH, x1_hbm)

    plsc.subcore_barrier()

    def div_pass(agg_hbm):
        def blk_body(blk, carry):
            row0 = s * (ROW_BLK * ROW_BLKS) + blk * ROW_BLK
            pltpu.sync_copy(acc.at[pl.ds(row0, ROW_BLK)], mw0)

            @plsc.parallel_loop(0, ROW_BLK, step=1, unroll=4)
            def _(r):
                for j in range(DH // 16):
                    num = mw0[r, pl.ds(j * 16, 16)]
                    den = mw0[r, pl.ds(DH + j * 16, 16)]
                    eb0[r, pl.ds(j * 16, 16)] = num / (den + 1e-16)
            pltpu.sync_copy(eb0, agg_hbm.at[pl.ds(row0, ROW_BLK)])
            return carry

        lax.fori_loop(0, ROW_BLKS, blk_body, 0)

    @pl.when(c == 0)
    def _():
        div_pass(agg0_hbm)

    @pl.when(c == 1)
    def _():
        div_pass(agg1_hbm)


def _sc_call(src_p, dst_p, x0, x1, ea, zeros):
    mesh = plsc.VectorSubcoreMesh(
        core_axis_name="c", subcore_axis_name="s", num_cores=NC,
        num_subcores=NS)
    f = functools.partial(
        pl.kernel,
        out_type=(
            jax.ShapeDtypeStruct((ACC_ROWS, DH), jnp.float32),
            jax.ShapeDtypeStruct((ACC_ROWS, DH), jnp.float32),
        ),
        mesh=mesh,
        compiler_params=pltpu.CompilerParams(use_tc_tiling_on_sc=False),
        scratch_types=[
            pltpu.VMEM_SHARED((ACC_ROWS, D), jnp.float32),
            pltpu.VMEM((CHUNK,), jnp.int32),
            pltpu.VMEM((CHUNK,), jnp.int32),
            pltpu.VMEM((CHUNK,), jnp.int32),
            pltpu.VMEM((CHUNK,), jnp.int32),
            pltpu.VMEM((CHUNK,), jnp.int32),
            pltpu.VMEM((CHUNK,), jnp.int32),
            pltpu.VMEM((CHUNK, DH), jnp.float32),
            pltpu.VMEM((CHUNK, DH), jnp.float32),
            pltpu.VMEM((CHUNK, DH), jnp.float32),
            pltpu.VMEM((CHUNK, DH), jnp.float32),
            pltpu.VMEM((CHUNK, D), jnp.float32),
            pltpu.VMEM((CHUNK, D), jnp.float32),
        ] + [pltpu.SemaphoreType.DMA] * 12,
    )(_sc_body)
    return f(src_p, dst_p, x0, x1, ea, zeros)


def _mlp_body(x0_ref, x1_ref, a0_ref, a1_ref, w1a_ref, w1b_ref,
              g_ref, b_ref, w2_ref, o_ref):
    h0 = a0_ref[...] + x0_ref[...]
    h1 = a1_ref[...] + x1_ref[...]
    z = (jnp.dot(h0, w1a_ref[...], preferred_element_type=jnp.float32)
         + jnp.dot(h1, w1b_ref[...], preferred_element_type=jnp.float32))
    mean = jnp.mean(z, axis=0, keepdims=True)
    zc = z - mean
    var = jnp.mean(zc * zc, axis=0, keepdims=True)
    zn = zc * lax.rsqrt(var + 1e-5) * g_ref[...] + b_ref[...]
    zr = jnp.maximum(zn, 0.0)
    o_ref[...] = jnp.dot(zr, w2_ref[...], preferred_element_type=jnp.float32)


def _mlp_call(x0, x1, a0, a1, w1a, w1b, gamma, beta, w2):
    return pl.pallas_call(
        _mlp_body,
        out_shape=jax.ShapeDtypeStruct((N_NODES, D), jnp.float32),
    )(x0, x1, a0, a1, w1a, w1b, gamma.reshape(1, 2 * D),
      beta.reshape(1, 2 * D), w2)


def kernel(x, edge_index, edge_attr, W_edge, W1, gamma, beta, W2):
    src = edge_index[0].astype(jnp.int32)
    dst = edge_index[1].astype(jnp.int32)
    src_p = jnp.concatenate([src, jnp.zeros((PAD,), jnp.int32)])
    dst_p = jnp.concatenate([dst, jnp.full((PAD,), N_NODES, jnp.int32)])
    x0 = x[:, :DH]
    x1 = x[:, DH:]
    zeros = jnp.zeros((ACC_ROWS, D), jnp.float32)

    ea = _ea_call(edge_attr, W_edge)
    agg0, agg1 = _sc_call(src_p, dst_p, x0, x1, ea, zeros)
    return _mlp_call(x0, x1, agg0[:N_NODES], agg1[:N_NODES],
                     W1[:DH], W1[DH:], gamma, beta, W2)
